# baseline (speedup 1.0000x reference)
import functools

import jax
import jax.numpy as jnp
from jax import lax
from jax.experimental import pallas as pl
from jax.experimental.pallas import tpu as pltpu
from jax.experimental.pallas import tpu_sc as plsc

N = 10000
E = 320000
D = 128
B = 128
EPS = 1e-5

NC, NS = 2, 16
CH = 128
NCH = 160
EPAD = NS * NCH * CH
NROW = N + 8
RPT = 624
TAIL = N - NS * RPT
ZR = 48

_mesh = plsc.VectorSubcoreMesh(
    core_axis_name="c", subcore_axis_name="s", num_cores=NC, num_subcores=NS)


def _zero_fill(buf, rows, width):
  z = jnp.zeros((16,), jnp.float32)

  def body(i, c):
    for j in range(width // 16):
      buf[i, pl.ds(j * 16, 16)] = z
    return c

  lax.fori_loop(0, rows, body, 0)


def _zero_acc(zbuf, acc, sid):
  _zero_fill(zbuf, ZR, D)
  for k in range(RPT // ZR):
    pltpu.sync_copy(zbuf, acc.at[pl.ds(sid * RPT + k * ZR, ZR)])

  @pl.when(sid == NS - 1)
  def _():
    pltpu.sync_copy(zbuf.at[pl.ds(0, TAIL)], acc.at[pl.ds(NS * RPT, TAIL)])


def _copy_out(zbuf, acc, out_hbm, cid, sid):
  for k in range(RPT // ZR):
    r0 = sid * RPT + k * ZR
    pltpu.sync_copy(acc.at[pl.ds(r0, ZR)], zbuf)
    pltpu.sync_copy(zbuf, out_hbm.at[cid, pl.ds(r0, ZR)])

  @pl.when(sid == NS - 1)
  def _():
    pltpu.sync_copy(acc.at[pl.ds(NS * RPT, TAIL)], zbuf.at[pl.ds(0, TAIL)])
    pltpu.sync_copy(zbuf.at[pl.ds(0, TAIL)],
                    out_hbm.at[cid, pl.ds(NS * RPT, TAIL)])


@functools.partial(
    pl.kernel,
    out_type=jax.ShapeDtypeStruct((NC, N, D), jnp.float32),
    mesh=_mesh,
    scratch_types=[
        pltpu.VMEM((CH,), jnp.int32),
        pltpu.VMEM((CH,), jnp.int32),
        pltpu.VMEM((CH, D), jnp.float32),
        pltpu.VMEM((ZR, D), jnp.float32),
        pltpu.VMEM_SHARED((NROW, D), jnp.float32),
        pltpu.SemaphoreType.DMA,
    ],
)
def _sc_scatter(y_hbm, src_hbm, dst_hbm, out_hbm,
                gidx, sidx, rows, zbuf, acc, sg):
  cid = lax.axis_index("c")
  sid = lax.axis_index("s")

  _zero_acc(zbuf, acc, sid)
  plsc.subcore_barrier()

  ebase = cid * EPAD + sid * (NCH * CH)

  def body(i, c):
    off = pl.multiple_of(ebase + i * CH, 8)
    pltpu.sync_copy(src_hbm.at[pl.ds(off, CH)], gidx)
    pltpu.async_copy(y_hbm.at[gidx], rows, sg).wait()
    pltpu.sync_copy(dst_hbm.at[pl.ds(off, CH)], sidx)
    pltpu.sync_copy(rows, acc.at[sidx], add=True)
    return c

  lax.fori_loop(0, NCH, body, 0)

  plsc.subcore_barrier()
  _copy_out(zbuf, acc, out_hbm, cid, sid)


@functools.partial(
    pl.kernel,
    out_type=jax.ShapeDtypeStruct((NC, N, D), jnp.float32),
    mesh=_mesh,
    scratch_types=[
        pltpu.VMEM((NCH, CH), jnp.int32),
        pltpu.VMEM((CH, D), jnp.float32),
        pltpu.VMEM((ZR, D), jnp.float32),
        pltpu.VMEM_SHARED((NROW, D), jnp.float32),
        pltpu.SemaphoreType.DMA,
    ],
)
def _sc_degree(dst_hbm, out_hbm, sslab, ones_v, zbuf, acc, sem):
  cid = lax.axis_index("c")
  sid = lax.axis_index("s")

  _zero_acc(zbuf, acc, sid)
  one = jnp.ones((16,), jnp.float32)

  def fill(i, c):
    for j in range(D // 16):
      ones_v[i, pl.ds(j * 16, 16)] = one
    return c

  lax.fori_loop(0, CH, fill, 0)
  pltpu.sync_copy(dst_hbm.at[cid, sid], sslab)
  plsc.subcore_barrier()

  DEPTH = 8
  for i in range(DEPTH):
    pltpu.async_copy(ones_v, acc.at[sslab.at[i]], sem, add=True)

  def body(i, c):
    pltpu.async_copy(ones_v, acc.at[sslab.at[i]], sem, add=True)
    pltpu.make_async_copy(ones_v, acc.at[sslab.at[0]], sem).wait()
    return c

  lax.fori_loop(DEPTH, NCH, body, 0)
  for i in range(DEPTH):
    pltpu.make_async_copy(ones_v, acc.at[sslab.at[0]], sem).wait()

  plsc.subcore_barrier()
  _copy_out(zbuf, acc, out_hbm, cid, sid)


RB = 1000
_BN_S = 1.0 / (1.0 + EPS) ** 0.5


def _dis(deg_ref):
  return lax.rsqrt(deg_ref[0][:, 0:1] + 1.0)


def _k0_body(deg_ref, x_ref, w_ref, y_ref):
  y_ref[0] = _dis(deg_ref) * jnp.dot(
      x_ref[0], w_ref[0], preferred_element_type=jnp.float32)


def _k12_body(deg_ref, acc_ref, y_ref, w_ref, b_ref, g_ref, bb_ref, ynext_ref):
  dis = _dis(deg_ref)
  out = dis * (acc_ref[0] + y_ref[0]) + b_ref[0]
  h = jnp.maximum(out * (_BN_S * g_ref[0]) + bb_ref[0], 0.0)
  ynext_ref[0] = dis * jnp.dot(h, w_ref[0], preferred_element_type=jnp.float32)


def _k3_body(deg_ref, acc_ref, y_ref, b_ref, x_ref, batch_ref,
             sums_ref, cnt_ref):
  h3 = _dis(deg_ref) * (acc_ref[0] + y_ref[0]) + b_ref[0] + x_ref[0]
  onehot = (batch_ref[0] == lax.broadcasted_iota(jnp.int32, (1, B), 1)
            ).astype(jnp.float32)
  dn = (((0,), (0,)), ((), ()))
  ps = lax.dot_general(onehot, h3, dn, preferred_element_type=jnp.float32)
  pc = lax.dot_general(onehot, jnp.ones((RB, D), jnp.float32), dn,
                       preferred_element_type=jnp.float32)

  @pl.when(pl.program_id(1) == 0)
  def _():
    sums_ref[0] = ps
    cnt_ref[0] = pc

  @pl.when(pl.program_id(1) > 0)
  def _():
    sums_ref[0] += ps
    cnt_ref[0] += pc


def _ln(x, g, b):
  m = jnp.mean(x, axis=-1, keepdims=True)
  v = jnp.mean((x - m) ** 2, axis=-1, keepdims=True)
  return (x - m) * lax.rsqrt(v + EPS) * g + b


def _cls_body(sums_ref, cnt_ref, t1_ref, t2_ref, ce_ref,
              w1_ref, b1_ref, g1_ref, bb1_ref,
              w2_ref, b2_ref, g2_ref, bb2_ref, w3_ref, b3_ref, out_ref):
  d1 = sums_ref[0] / jnp.maximum(cnt_ref[0], 1.0)
  d2 = sums_ref[1] / jnp.maximum(cnt_ref[1], 1.0)
  fused = jnp.concatenate(
      [d1, d2, t1_ref[...], t2_ref[...], ce_ref[...]], axis=1)
  h = jnp.dot(fused, w1_ref[...], preferred_element_type=jnp.float32)
  h = jnp.maximum(_ln(h + b1_ref[...], g1_ref[...], bb1_ref[...]), 0.0)
  h = jnp.dot(h, w2_ref[...], preferred_element_type=jnp.float32)
  h = jnp.maximum(_ln(h + b2_ref[...], g2_ref[...], bb2_ref[...]), 0.0)
  out_ref[...] = jnp.dot(h, w3_ref[...],
                         preferred_element_type=jnp.float32) + b3_ref[...]


def _row_spec(width):
  return pl.BlockSpec((1, RB, width), lambda g, i: (g, i, 0))


_W_SPEC = pl.BlockSpec((1, D, D), lambda g, i: (g, 0, 0))
_V_SPEC = pl.BlockSpec((1, 1, D), lambda g, i: (g, 0, 0))
_GRID = (NC, N // RB)


def _dense_stage0(deg, x, w):
  return pl.pallas_call(
      _k0_body,
      grid=_GRID,
      in_specs=[_row_spec(D), _row_spec(D), _W_SPEC],
      out_specs=_row_spec(D),
      out_shape=jax.ShapeDtypeStruct((NC, N, D), jnp.float32),
  )(deg, x, w)


def _dense_stage12(deg, acc, y, w, b, g, bb):
  return pl.pallas_call(
      _k12_body,
      grid=_GRID,
      in_specs=[_row_spec(D), _row_spec(D), _row_spec(D), _W_SPEC,
                _V_SPEC, _V_SPEC, _V_SPEC],
      out_specs=_row_spec(D),
      out_shape=jax.ShapeDtypeStruct((NC, N, D), jnp.float32),
  )(deg, acc, y, w, b, g, bb)


def _dense_stage3(deg, acc, y, b, x, batch):
  pool_spec = pl.BlockSpec((1, B, D), lambda g, i: (g, 0, 0))
  return pl.pallas_call(
      _k3_body,
      grid=_GRID,
      in_specs=[_row_spec(D), _row_spec(D), _row_spec(D), _V_SPEC,
                _row_spec(D), _row_spec(1)],
      out_specs=[pool_spec, pool_spec],
      out_shape=[jax.ShapeDtypeStruct((NC, B, D), jnp.float32),
                 jax.ShapeDtypeStruct((NC, B, D), jnp.float32)],
  )(deg, acc, y, b, x, batch)


def _classifier(sums, cnt, t1, t2, ce, p):
  w3 = jnp.zeros((D, D), jnp.float32).at[:, :2].set(p['W3'])
  b3 = jnp.zeros((1, D), jnp.float32).at[0, :2].set(p['b3'])
  out = pl.pallas_call(
      _cls_body,
      out_shape=jax.ShapeDtypeStruct((B, D), jnp.float32),
  )(sums, cnt, t1, t2, ce,
    p['W1'], p['b1'].reshape(1, -1), p['ln1_g'].reshape(1, -1),
    p['ln1_b'].reshape(1, -1),
    p['W2'], p['b2'].reshape(1, -1), p['ln2_g'].reshape(1, -1),
    p['ln2_b'].reshape(1, -1), w3, b3)
  return out[:, :2]


def kernel(x1, edge_index1, batch1, x2, edge_index2, batch2,
           target1, target2, cell_expr, params):
  pad = EPAD - E
  zpad = jnp.zeros((pad,), jnp.int32)
  npad = jnp.full((pad,), N, jnp.int32)
  src = jnp.concatenate([edge_index1[0], zpad, edge_index2[0] + N, zpad]
                        ).reshape(NC, NS, NCH, CH)
  dst = jnp.concatenate([edge_index1[1], npad, edge_index2[1], npad]
                        ).reshape(NC, NS, NCH, CH)
  src_flat = src.reshape(-1)
  dst_flat = dst.reshape(-1)
  x = jnp.stack([x1, x2])
  batch = jnp.stack([batch1, batch2]).reshape(NC, N, 1)
  p1, p2 = params['g1'], params['g2']

  def stk(name):
    v = jnp.stack([p1[name], p2[name]])
    return v.reshape(NC, 1, D) if v.ndim == 2 else v

  deg = _sc_degree(dst)

  y = _dense_stage0(deg, x, stk('W1'))
  acc = _sc_scatter(y.reshape(NC * N, D), src_flat, dst_flat)
  y = _dense_stage12(deg, acc, y, stk('W2'), stk('b1'), stk('bn1_g'),
                     stk('bn1_b'))
  acc = _sc_scatter(y.reshape(NC * N, D), src_flat, dst_flat)
  y = _dense_stage12(deg, acc, y, stk('W3'), stk('b2'), stk('bn2_g'),
                     stk('bn2_b'))
  acc = _sc_scatter(y.reshape(NC * N, D), src_flat, dst_flat)
  sums, cnt = _dense_stage3(deg, acc, y, stk('b3'), x, batch)

  return _classifier(sums, cnt, target1, target2, cell_expr, params['cls'])


# speedup vs baseline: 1.7198x; 1.7198x over previous
import functools

import jax
import jax.numpy as jnp
from jax import lax
from jax.experimental import pallas as pl
from jax.experimental.pallas import tpu as pltpu
from jax.experimental.pallas import tpu_sc as plsc

N = 10000
E = 320000
D = 128
B = 128
EPS = 1e-5

NC, NS = 2, 16
CH = 128
NCH = 160
EPAD = NS * NCH * CH
NROW = N + 8
RPT = 624
TAIL = N - NS * RPT
ZR = 48

_mesh = plsc.VectorSubcoreMesh(
    core_axis_name="c", subcore_axis_name="s", num_cores=NC, num_subcores=NS)


def _zero_fill(buf, rows, width):
  z = jnp.zeros((16,), jnp.float32)

  def body(i, c):
    for j in range(width // 16):
      buf[i, pl.ds(j * 16, 16)] = z
    return c

  lax.fori_loop(0, rows, body, 0)


def _zero_acc(zbuf, acc, sid):
  _zero_fill(zbuf, ZR, D)
  for k in range(RPT // ZR):
    pltpu.sync_copy(zbuf, acc.at[pl.ds(sid * RPT + k * ZR, ZR)])

  @pl.when(sid == NS - 1)
  def _():
    pltpu.sync_copy(zbuf.at[pl.ds(0, TAIL)], acc.at[pl.ds(NS * RPT, TAIL)])


def _copy_out(zbuf, acc, out_hbm, cid, sid):
  for k in range(RPT // ZR):
    r0 = sid * RPT + k * ZR
    pltpu.sync_copy(acc.at[pl.ds(r0, ZR)], zbuf)
    pltpu.sync_copy(zbuf, out_hbm.at[cid, pl.ds(r0, ZR)])

  @pl.when(sid == NS - 1)
  def _():
    pltpu.sync_copy(acc.at[pl.ds(NS * RPT, TAIL)], zbuf.at[pl.ds(0, TAIL)])
    pltpu.sync_copy(zbuf.at[pl.ds(0, TAIL)],
                    out_hbm.at[cid, pl.ds(NS * RPT, TAIL)])


@functools.partial(
    pl.kernel,
    out_type=jax.ShapeDtypeStruct((NC, N, D), jnp.float32),
    mesh=_mesh,
    scratch_types=[
        pltpu.VMEM((CH,), jnp.int32),
        pltpu.VMEM((CH,), jnp.int32),
        pltpu.VMEM((CH, D), jnp.float32),
        pltpu.VMEM((ZR, D), jnp.float32),
        pltpu.VMEM_SHARED((NROW, D), jnp.float32),
        pltpu.SemaphoreType.DMA,
    ],
)
def _sc_scatter(y_hbm, src_hbm, dst_hbm, out_hbm,
                gidx, sidx, rows, zbuf, acc, sg):
  cid = lax.axis_index("c")
  sid = lax.axis_index("s")

  _zero_acc(zbuf, acc, sid)
  plsc.subcore_barrier()

  ebase = cid * EPAD + sid * (NCH * CH)

  def body(i, c):
    off = pl.multiple_of(ebase + i * CH, 8)
    pltpu.sync_copy(src_hbm.at[pl.ds(off, CH)], gidx)
    pltpu.async_copy(y_hbm.at[gidx], rows, sg).wait()
    pltpu.sync_copy(dst_hbm.at[pl.ds(off, CH)], sidx)
    pltpu.sync_copy(rows, acc.at[sidx], add=True)
    return c

  lax.fori_loop(0, NCH, body, 0)

  plsc.subcore_barrier()
  _copy_out(zbuf, acc, out_hbm, cid, sid)


@functools.partial(
    pl.kernel,
    out_type=jax.ShapeDtypeStruct((NC, N, D), jnp.float32),
    mesh=_mesh,
    scratch_types=[
        pltpu.VMEM((NCH, CH), jnp.int32),
        pltpu.VMEM((CH, D), jnp.float32),
        pltpu.VMEM((ZR, D), jnp.float32),
        pltpu.VMEM_SHARED((NROW, D), jnp.float32),
        pltpu.SemaphoreType.DMA,
    ],
)
def _sc_degree(dst_hbm, out_hbm, sslab, ones_v, zbuf, acc, sem):
  cid = lax.axis_index("c")
  sid = lax.axis_index("s")

  _zero_acc(zbuf, acc, sid)
  one = jnp.ones((16,), jnp.float32)

  def fill(i, c):
    for j in range(D // 16):
      ones_v[i, pl.ds(j * 16, 16)] = one
    return c

  lax.fori_loop(0, CH, fill, 0)
  pltpu.sync_copy(dst_hbm.at[cid, sid], sslab)
  plsc.subcore_barrier()

  DEPTH = 8
  for i in range(DEPTH):
    pltpu.async_copy(ones_v, acc.at[sslab.at[i]], sem, add=True)

  def body(i, c):
    pltpu.async_copy(ones_v, acc.at[sslab.at[i]], sem, add=True)
    pltpu.make_async_copy(ones_v, acc.at[sslab.at[0]], sem).wait()
    return c

  lax.fori_loop(DEPTH, NCH, body, 0)
  for i in range(DEPTH):
    pltpu.make_async_copy(ones_v, acc.at[sslab.at[0]], sem).wait()

  plsc.subcore_barrier()
  _copy_out(zbuf, acc, out_hbm, cid, sid)


RB = 1000
_BN_S = 1.0 / (1.0 + EPS) ** 0.5


def _dis(deg_ref):
  return lax.rsqrt(deg_ref[0][:, 0:1] + 1.0)


def _k0_body(deg_ref, x_ref, w_ref, y_ref):
  y_ref[0] = _dis(deg_ref) * jnp.dot(
      x_ref[0], w_ref[0], preferred_element_type=jnp.float32)


def _k12_body(deg_ref, acc_ref, y_ref, w_ref, b_ref, g_ref, bb_ref, ynext_ref):
  dis = _dis(deg_ref)
  out = dis * (acc_ref[0] + y_ref[0]) + b_ref[0]
  h = jnp.maximum(out * (_BN_S * g_ref[0]) + bb_ref[0], 0.0)
  ynext_ref[0] = dis * jnp.dot(h, w_ref[0], preferred_element_type=jnp.float32)


def _k3_body(deg_ref, acc_ref, y_ref, b_ref, x_ref, batch_ref,
             sums_ref, cnt_ref):
  h3 = _dis(deg_ref) * (acc_ref[0] + y_ref[0]) + b_ref[0] + x_ref[0]
  onehot = (batch_ref[0] == lax.broadcasted_iota(jnp.int32, (1, B), 1)
            ).astype(jnp.float32)
  dn = (((0,), (0,)), ((), ()))
  ps = lax.dot_general(onehot, h3, dn, preferred_element_type=jnp.float32)
  pc = lax.dot_general(onehot, jnp.ones((RB, D), jnp.float32), dn,
                       preferred_element_type=jnp.float32)

  @pl.when(pl.program_id(1) == 0)
  def _():
    sums_ref[0] = ps
    cnt_ref[0] = pc

  @pl.when(pl.program_id(1) > 0)
  def _():
    sums_ref[0] += ps
    cnt_ref[0] += pc


def _ln(x, g, b):
  m = jnp.mean(x, axis=-1, keepdims=True)
  v = jnp.mean((x - m) ** 2, axis=-1, keepdims=True)
  return (x - m) * lax.rsqrt(v + EPS) * g + b


def _cls_body(sums_ref, cnt_ref, t1_ref, t2_ref, ce_ref,
              w1_ref, b1_ref, g1_ref, bb1_ref,
              w2_ref, b2_ref, g2_ref, bb2_ref, w3_ref, b3_ref, out_ref):
  d1 = sums_ref[0] / jnp.maximum(cnt_ref[0], 1.0)
  d2 = sums_ref[1] / jnp.maximum(cnt_ref[1], 1.0)
  fused = jnp.concatenate(
      [d1, d2, t1_ref[...], t2_ref[...], ce_ref[...]], axis=1)
  h = jnp.dot(fused, w1_ref[...], preferred_element_type=jnp.float32)
  h = jnp.maximum(_ln(h + b1_ref[...], g1_ref[...], bb1_ref[...]), 0.0)
  h = jnp.dot(h, w2_ref[...], preferred_element_type=jnp.float32)
  h = jnp.maximum(_ln(h + b2_ref[...], g2_ref[...], bb2_ref[...]), 0.0)
  out_ref[...] = jnp.dot(h, w3_ref[...],
                         preferred_element_type=jnp.float32) + b3_ref[...]


def _row_spec(width):
  return pl.BlockSpec((1, RB, width), lambda g, i: (g, i, 0))


_W_SPEC = pl.BlockSpec((1, D, D), lambda g, i: (g, 0, 0))
_V_SPEC = pl.BlockSpec((1, 1, D), lambda g, i: (g, 0, 0))
_GRID = (NC, N // RB)


def _dense_stage0(deg, x, w):
  return pl.pallas_call(
      _k0_body,
      grid=_GRID,
      in_specs=[_row_spec(D), _row_spec(D), _W_SPEC],
      out_specs=_row_spec(D),
      out_shape=jax.ShapeDtypeStruct((NC, N, D), jnp.float32),
  )(deg, x, w)


def _dense_stage12(deg, acc, y, w, b, g, bb):
  return pl.pallas_call(
      _k12_body,
      grid=_GRID,
      in_specs=[_row_spec(D), _row_spec(D), _row_spec(D), _W_SPEC,
                _V_SPEC, _V_SPEC, _V_SPEC],
      out_specs=_row_spec(D),
      out_shape=jax.ShapeDtypeStruct((NC, N, D), jnp.float32),
  )(deg, acc, y, w, b, g, bb)


def _dense_stage3(deg, acc, y, b, x, batch):
  pool_spec = pl.BlockSpec((1, B, D), lambda g, i: (g, 0, 0))
  return pl.pallas_call(
      _k3_body,
      grid=_GRID,
      in_specs=[_row_spec(D), _row_spec(D), _row_spec(D), _V_SPEC,
                _row_spec(D), _row_spec(1)],
      out_specs=[pool_spec, pool_spec],
      out_shape=[jax.ShapeDtypeStruct((NC, B, D), jnp.float32),
                 jax.ShapeDtypeStruct((NC, B, D), jnp.float32)],
  )(deg, acc, y, b, x, batch)


def _classifier(sums, cnt, t1, t2, ce, p):
  w3 = jnp.zeros((D, D), jnp.float32).at[:, :2].set(p['W3'])
  b3 = jnp.zeros((1, D), jnp.float32).at[0, :2].set(p['b3'])
  out = pl.pallas_call(
      _cls_body,
      out_shape=jax.ShapeDtypeStruct((B, D), jnp.float32),
  )(sums, cnt, t1, t2, ce,
    p['W1'], p['b1'].reshape(1, -1), p['ln1_g'].reshape(1, -1),
    p['ln1_b'].reshape(1, -1),
    p['W2'], p['b2'].reshape(1, -1), p['ln2_g'].reshape(1, -1),
    p['ln2_b'].reshape(1, -1), w3, b3)
  return out[:, :2]


def kernel(x1, edge_index1, batch1, x2, edge_index2, batch2,
           target1, target2, cell_expr, params):
  pad = EPAD - E
  pidx = jnp.arange(pad, dtype=jnp.int32)
  zsrc = 2 * N + (pidx % 8)
  sdst = pidx % N
  ddst = N + (pidx % 8)
  src_flat = jnp.concatenate(
      [edge_index1[0], zsrc, edge_index2[0] + N, zsrc])
  dst_flat = jnp.concatenate([edge_index1[1], sdst, edge_index2[1], sdst])
  dst = jnp.concatenate([edge_index1[1], ddst, edge_index2[1], ddst]
                        ).reshape(NC, NS, NCH, CH)
  x = jnp.stack([x1, x2])
  batch = jnp.stack([batch1, batch2]).reshape(NC, N, 1)
  p1, p2 = params['g1'], params['g2']

  def stk(name):
    v = jnp.stack([p1[name], p2[name]])
    return v.reshape(NC, 1, D) if v.ndim == 2 else v

  deg = _sc_degree(dst)

  y = _dense_stage0(deg, x, stk('W1'))
  acc = _sc_scatter(
      jnp.concatenate([y.reshape(NC * N, D), jnp.zeros((8, D), jnp.float32)]),
      src_flat, dst_flat)
  y = _dense_stage12(deg, acc, y, stk('W2'), stk('b1'), stk('bn1_g'),
                     stk('bn1_b'))
  acc = _sc_scatter(
      jnp.concatenate([y.reshape(NC * N, D), jnp.zeros((8, D), jnp.float32)]),
      src_flat, dst_flat)
  y = _dense_stage12(deg, acc, y, stk('W3'), stk('b2'), stk('bn2_g'),
                     stk('bn2_b'))
  acc = _sc_scatter(
      jnp.concatenate([y.reshape(NC * N, D), jnp.zeros((8, D), jnp.float32)]),
      src_flat, dst_flat)
  sums, cnt = _dense_stage3(deg, acc, y, stk('b3'), x, batch)

  return _classifier(sums, cnt, target1, target2, cell_expr, params['cls'])


# speedup vs baseline: 2.7061x; 1.5735x over previous
import functools

import jax
import jax.numpy as jnp
from jax import lax
from jax.experimental import pallas as pl
from jax.experimental.pallas import tpu as pltpu
from jax.experimental.pallas import tpu_sc as plsc

N = 10000
E = 320000
D = 128
B = 128
EPS = 1e-5

NC, NS = 2, 16
CH = 128
NCH = 160
EPAD = NS * NCH * CH
NROW = N + 8
RPT = 624
TAIL = N - NS * RPT
ZR = 48

_mesh = plsc.VectorSubcoreMesh(
    core_axis_name="c", subcore_axis_name="s", num_cores=NC, num_subcores=NS)


def _zero_fill(buf, rows, width):
  z = jnp.zeros((16,), jnp.float32)

  def body(i, c):
    for j in range(width // 16):
      buf[i, pl.ds(j * 16, 16)] = z
    return c

  lax.fori_loop(0, rows, body, 0)


def _zero_acc(zbuf, acc, sid):
  _zero_fill(zbuf, ZR, D)
  for k in range(RPT // ZR):
    pltpu.sync_copy(zbuf, acc.at[pl.ds(sid * RPT + k * ZR, ZR)])

  @pl.when(sid == NS - 1)
  def _():
    pltpu.sync_copy(zbuf.at[pl.ds(0, TAIL)], acc.at[pl.ds(NS * RPT, TAIL)])


def _copy_out(zbuf, acc, out_hbm, cid, sid):
  for k in range(RPT // ZR):
    r0 = sid * RPT + k * ZR
    pltpu.sync_copy(acc.at[pl.ds(r0, ZR)], zbuf)
    pltpu.sync_copy(zbuf, out_hbm.at[cid, pl.ds(r0, ZR)])

  @pl.when(sid == NS - 1)
  def _():
    pltpu.sync_copy(acc.at[pl.ds(NS * RPT, TAIL)], zbuf.at[pl.ds(0, TAIL)])
    pltpu.sync_copy(zbuf.at[pl.ds(0, TAIL)],
                    out_hbm.at[cid, pl.ds(NS * RPT, TAIL)])


@functools.partial(
    pl.kernel,
    out_type=jax.ShapeDtypeStruct((NC, N, D), jnp.float32),
    mesh=_mesh,
    scratch_types=[
        pltpu.VMEM((2, CH), jnp.int32),
        pltpu.VMEM((2, CH), jnp.int32),
        pltpu.VMEM((2, CH), jnp.int32),
        pltpu.VMEM((2, CH), jnp.int32),
        pltpu.VMEM((CH, D), jnp.float32),
        pltpu.VMEM((CH, D), jnp.float32),
        pltpu.VMEM((ZR, D), jnp.float32),
        pltpu.VMEM_SHARED((NROW, D), jnp.float32),
        pltpu.SemaphoreType.DMA,
        pltpu.SemaphoreType.DMA,
        pltpu.SemaphoreType.DMA,
        pltpu.SemaphoreType.DMA,
        pltpu.SemaphoreType.DMA,
        pltpu.SemaphoreType.DMA,
        pltpu.SemaphoreType.DMA,
        pltpu.SemaphoreType.DMA,
    ],
)
def _sc_scatter(y_hbm, e_hbm, out_hbm,
                ib0, ib1, ib2, ib3, rows_a, rows_b, zbuf, acc,
                si0, si1, si2, si3, sga, sgb, ssa, ssb):
  cid = lax.axis_index("c")
  sid = lax.axis_index("s")

  ibufs = (ib0, ib1, ib2, ib3)
  isems = (si0, si1, si2, si3)
  rbufs = (rows_a, rows_b)
  gsems = (sga, sgb)
  ssems = (ssa, ssb)

  _zero_acc(zbuf, acc, sid)

  def start_idx(i, c):
    pltpu.async_copy(e_hbm.at[cid, sid, i], ibufs[c % 4], isems[c % 4])

  def wait_idx(c):
    pltpu.make_async_copy(
        e_hbm.at[cid, sid, 0], ibufs[c % 4], isems[c % 4]).wait()

  def start_g(c):
    pltpu.async_copy(
        y_hbm.at[ibufs[c % 4].at[0]], rbufs[c % 2], gsems[c % 2])

  def wait_g(c):
    pltpu.make_async_copy(
        y_hbm.at[ibufs[c % 4].at[0]], rbufs[c % 2], gsems[c % 2]).wait()

  def start_s(c):
    pltpu.async_copy(
        rbufs[c % 2], acc.at[ibufs[c % 4].at[1]], ssems[c % 2], add=True)

  def wait_s(c):
    pltpu.make_async_copy(
        rbufs[c % 2], acc.at[ibufs[c % 4].at[1]], ssems[c % 2]).wait()

  def step(i, c, with_s_prev=True, with_idx=True, with_g=True):
    wait_g(c)
    if with_s_prev:
      wait_s(c + 1)
    if with_idx:
      start_idx(i + 3, c + 3)
    if with_g:
      wait_idx(c + 1)
      start_g(c + 1)
    start_s(c)

  start_idx(0, 0)
  start_idx(1, 1)
  start_idx(2, 2)
  wait_idx(0)
  start_g(0)
  step(0, 0, with_s_prev=False)
  step(1, 1)

  def quad(k, c):
    i = 4 * k + 2
    step(i, 2)
    step(i + 1, 3)
    step(i + 2, 0)
    step(i + 3, 1)
    return c

  lax.fori_loop(0, (NCH - 8) // 4, quad, 0)

  for i in range(NCH - 6, NCH):
    step(i, i % 4, with_idx=(i + 3 < NCH), with_g=(i + 1 < NCH))
  wait_s(NCH - 1)

  plsc.subcore_barrier()
  _copy_out(zbuf, acc, out_hbm, cid, sid)


@functools.partial(
    pl.kernel,
    out_type=jax.ShapeDtypeStruct((NC, N, D), jnp.float32),
    mesh=_mesh,
    scratch_types=[
        pltpu.VMEM((NCH, CH), jnp.int32),
        pltpu.VMEM((CH, D), jnp.float32),
        pltpu.VMEM((ZR, D), jnp.float32),
        pltpu.VMEM_SHARED((NROW, D), jnp.float32),
        pltpu.SemaphoreType.DMA,
    ],
)
def _sc_degree(dst_hbm, out_hbm, sslab, ones_v, zbuf, acc, sem):
  cid = lax.axis_index("c")
  sid = lax.axis_index("s")

  _zero_acc(zbuf, acc, sid)
  one = jnp.ones((16,), jnp.float32)

  def fill(i, c):
    for j in range(D // 16):
      ones_v[i, pl.ds(j * 16, 16)] = one
    return c

  lax.fori_loop(0, CH, fill, 0)
  pltpu.sync_copy(dst_hbm.at[cid, sid], sslab)
  plsc.subcore_barrier()

  DEPTH = 8
  for i in range(DEPTH):
    pltpu.async_copy(ones_v, acc.at[sslab.at[i]], sem, add=True)

  def body(i, c):
    pltpu.async_copy(ones_v, acc.at[sslab.at[i]], sem, add=True)
    pltpu.make_async_copy(ones_v, acc.at[sslab.at[0]], sem).wait()
    return c

  lax.fori_loop(DEPTH, NCH, body, 0)
  for i in range(DEPTH):
    pltpu.make_async_copy(ones_v, acc.at[sslab.at[0]], sem).wait()

  plsc.subcore_barrier()
  _copy_out(zbuf, acc, out_hbm, cid, sid)


RB = 1000
_BN_S = 1.0 / (1.0 + EPS) ** 0.5


def _dis(deg_ref):
  return lax.rsqrt(deg_ref[0][:, 0:1] + 1.0)


def _k0_body(deg_ref, x_ref, w_ref, y_ref):
  y_ref[0] = _dis(deg_ref) * jnp.dot(
      x_ref[0], w_ref[0], preferred_element_type=jnp.float32)


def _k12_body(deg_ref, acc_ref, y_ref, w_ref, b_ref, g_ref, bb_ref, ynext_ref):
  dis = _dis(deg_ref)
  out = dis * (acc_ref[0] + y_ref[0]) + b_ref[0]
  h = jnp.maximum(out * (_BN_S * g_ref[0]) + bb_ref[0], 0.0)
  ynext_ref[0] = dis * jnp.dot(h, w_ref[0], preferred_element_type=jnp.float32)


def _k3_body(deg_ref, acc_ref, y_ref, b_ref, x_ref, batch_ref,
             sums_ref, cnt_ref):
  h3 = _dis(deg_ref) * (acc_ref[0] + y_ref[0]) + b_ref[0] + x_ref[0]
  onehot = (batch_ref[0] == lax.broadcasted_iota(jnp.int32, (1, B), 1)
            ).astype(jnp.float32)
  dn = (((0,), (0,)), ((), ()))
  ps = lax.dot_general(onehot, h3, dn, preferred_element_type=jnp.float32)
  pc = lax.dot_general(onehot, jnp.ones((RB, D), jnp.float32), dn,
                       preferred_element_type=jnp.float32)

  @pl.when(pl.program_id(1) == 0)
  def _():
    sums_ref[0] = ps
    cnt_ref[0] = pc

  @pl.when(pl.program_id(1) > 0)
  def _():
    sums_ref[0] += ps
    cnt_ref[0] += pc


def _ln(x, g, b):
  m = jnp.mean(x, axis=-1, keepdims=True)
  v = jnp.mean((x - m) ** 2, axis=-1, keepdims=True)
  return (x - m) * lax.rsqrt(v + EPS) * g + b


def _cls_body(sums_ref, cnt_ref, t1_ref, t2_ref, ce_ref,
              w1_ref, b1_ref, g1_ref, bb1_ref,
              w2_ref, b2_ref, g2_ref, bb2_ref, w3_ref, b3_ref, out_ref):
  d1 = sums_ref[0] / jnp.maximum(cnt_ref[0], 1.0)
  d2 = sums_ref[1] / jnp.maximum(cnt_ref[1], 1.0)
  fused = jnp.concatenate(
      [d1, d2, t1_ref[...], t2_ref[...], ce_ref[...]], axis=1)
  h = jnp.dot(fused, w1_ref[...], preferred_element_type=jnp.float32)
  h = jnp.maximum(_ln(h + b1_ref[...], g1_ref[...], bb1_ref[...]), 0.0)
  h = jnp.dot(h, w2_ref[...], preferred_element_type=jnp.float32)
  h = jnp.maximum(_ln(h + b2_ref[...], g2_ref[...], bb2_ref[...]), 0.0)
  out_ref[...] = jnp.dot(h, w3_ref[...],
                         preferred_element_type=jnp.float32) + b3_ref[...]


def _row_spec(width):
  return pl.BlockSpec((1, RB, width), lambda g, i: (g, i, 0))


_W_SPEC = pl.BlockSpec((1, D, D), lambda g, i: (g, 0, 0))
_V_SPEC = pl.BlockSpec((1, 1, D), lambda g, i: (g, 0, 0))
_GRID = (NC, N // RB)


def _dense_stage0(deg, x, w):
  return pl.pallas_call(
      _k0_body,
      grid=_GRID,
      in_specs=[_row_spec(D), _row_spec(D), _W_SPEC],
      out_specs=_row_spec(D),
      out_shape=jax.ShapeDtypeStruct((NC, N, D), jnp.float32),
  )(deg, x, w)


def _dense_stage12(deg, acc, y, w, b, g, bb):
  return pl.pallas_call(
      _k12_body,
      grid=_GRID,
      in_specs=[_row_spec(D), _row_spec(D), _row_spec(D), _W_SPEC,
                _V_SPEC, _V_SPEC, _V_SPEC],
      out_specs=_row_spec(D),
      out_shape=jax.ShapeDtypeStruct((NC, N, D), jnp.float32),
  )(deg, acc, y, w, b, g, bb)


def _dense_stage3(deg, acc, y, b, x, batch):
  pool_spec = pl.BlockSpec((1, B, D), lambda g, i: (g, 0, 0))
  return pl.pallas_call(
      _k3_body,
      grid=_GRID,
      in_specs=[_row_spec(D), _row_spec(D), _row_spec(D), _V_SPEC,
                _row_spec(D), _row_spec(1)],
      out_specs=[pool_spec, pool_spec],
      out_shape=[jax.ShapeDtypeStruct((NC, B, D), jnp.float32),
                 jax.ShapeDtypeStruct((NC, B, D), jnp.float32)],
  )(deg, acc, y, b, x, batch)


def _classifier(sums, cnt, t1, t2, ce, p):
  w3 = jnp.zeros((D, D), jnp.float32).at[:, :2].set(p['W3'])
  b3 = jnp.zeros((1, D), jnp.float32).at[0, :2].set(p['b3'])
  out = pl.pallas_call(
      _cls_body,
      out_shape=jax.ShapeDtypeStruct((B, D), jnp.float32),
  )(sums, cnt, t1, t2, ce,
    p['W1'], p['b1'].reshape(1, -1), p['ln1_g'].reshape(1, -1),
    p['ln1_b'].reshape(1, -1),
    p['W2'], p['b2'].reshape(1, -1), p['ln2_g'].reshape(1, -1),
    p['ln2_b'].reshape(1, -1), w3, b3)
  return out[:, :2]


def kernel(x1, edge_index1, batch1, x2, edge_index2, batch2,
           target1, target2, cell_expr, params):
  pad = EPAD - E
  pidx = jnp.arange(pad, dtype=jnp.int32)
  zsrc = 2 * N + (pidx % 8)
  sdst = pidx % N
  ddst = N + (pidx % 8)
  src_flat = jnp.concatenate(
      [edge_index1[0], zsrc, edge_index2[0] + N, zsrc])
  dst_flat = jnp.concatenate([edge_index1[1], sdst, edge_index2[1], sdst])
  dst = jnp.concatenate([edge_index1[1], ddst, edge_index2[1], ddst]
                        ).reshape(NC, NS, NCH, CH)
  edges = jnp.stack([src_flat.reshape(NC, NS, NCH, CH),
                     dst_flat.reshape(NC, NS, NCH, CH)], axis=3)
  x = jnp.stack([x1, x2])
  batch = jnp.stack([batch1, batch2]).reshape(NC, N, 1)
  p1, p2 = params['g1'], params['g2']

  def stk(name):
    v = jnp.stack([p1[name], p2[name]])
    return v.reshape(NC, 1, D) if v.ndim == 2 else v

  deg = _sc_degree(dst)

  y = _dense_stage0(deg, x, stk('W1'))
  acc = _sc_scatter(
      jnp.concatenate([y.reshape(NC * N, D), jnp.zeros((8, D), jnp.float32)]),
      edges)
  y = _dense_stage12(deg, acc, y, stk('W2'), stk('b1'), stk('bn1_g'),
                     stk('bn1_b'))
  acc = _sc_scatter(
      jnp.concatenate([y.reshape(NC * N, D), jnp.zeros((8, D), jnp.float32)]),
      edges)
  y = _dense_stage12(deg, acc, y, stk('W3'), stk('b2'), stk('bn2_g'),
                     stk('bn2_b'))
  acc = _sc_scatter(
      jnp.concatenate([y.reshape(NC * N, D), jnp.zeros((8, D), jnp.float32)]),
      edges)
  sums, cnt = _dense_stage3(deg, acc, y, stk('b3'), x, batch)

  return _classifier(sums, cnt, target1, target2, cell_expr, params['cls'])
